# Initial kernel scaffold; baseline (speedup 1.0000x reference)
import jax
import jax.numpy as jnp
from jax.experimental import pallas as pl


def kernel(velocity_bins, table):
    raise NotImplementedError("write your pallas kernel here")


# speedup vs baseline: 1.6225x; 1.6225x over previous
import functools

import jax
import jax.numpy as jnp
from jax import lax
from jax.experimental import pallas as pl
from jax.experimental.pallas import tpu as pltpu
from jax.experimental.pallas import tpu_sc as plsc

NUM_BINS = 32
EMBED_DIM = 64
CHUNK = 128


@functools.lru_cache(maxsize=None)
def _sc_gather(n_total: int):
    info = plsc.get_sparse_core_info()
    nc, ns = info.num_cores, info.num_subcores
    nw = nc * ns
    per_w = n_total // nw
    assert per_w * nw == n_total and per_w % CHUNK == 0
    n_chunks = per_w // CHUNK
    mesh = plsc.VectorSubcoreMesh(core_axis_name="c", subcore_axis_name="s")

    @functools.partial(
        pl.kernel,
        out_type=jax.ShapeDtypeStruct((n_total, EMBED_DIM), jnp.float32),
        mesh=mesh,
        scratch_types=[
            pltpu.VMEM((n_chunks, CHUNK), jnp.int32),
            pltpu.VMEM((CHUNK, EMBED_DIM), jnp.float32),
            pltpu.SemaphoreType.DMA,
        ],
        compiler_params=pltpu.CompilerParams(use_tc_tiling_on_sc=False),
    )
    def k(idx_hbm, table_hbm, out_hbm, idx_v, rows_v, sem):
        wid = lax.axis_index("s") * nc + lax.axis_index("c")
        row0 = wid * n_chunks
        pltpu.sync_copy(idx_hbm.at[pl.ds(row0, n_chunks)], idx_v)

        def chunk_body(j, carry):
            pltpu.async_copy(table_hbm.at[idx_v.at[j]], rows_v, sem).wait()
            pltpu.sync_copy(rows_v, out_hbm.at[pl.ds((row0 + j) * CHUNK, CHUNK)])
            return carry

        lax.fori_loop(0, n_chunks, chunk_body, 0)

    return k


def kernel(velocity_bins, table):
    b, s = velocity_bins.shape
    n = b * s
    idx2d = velocity_bins.astype(jnp.int32).reshape(n // CHUNK, CHUNK)
    out = _sc_gather(n)(idx2d, table)
    return out.reshape(b, s, EMBED_DIM)


# speedup vs baseline: 1.6363x; 1.0085x over previous
import functools

import jax
import jax.numpy as jnp
from jax import lax
from jax.experimental import pallas as pl
from jax.experimental.pallas import tpu as pltpu
from jax.experimental.pallas import tpu_sc as plsc

NUM_BINS = 32
EMBED_DIM = 64
CHUNK = 128
K = 5


@functools.lru_cache(maxsize=None)
def _sc_gather(n_total: int):
    info = plsc.get_sparse_core_info()
    nc, ns = info.num_cores, info.num_subcores
    nw = nc * ns
    per_w = n_total // nw
    assert per_w * nw == n_total and per_w % (CHUNK * K) == 0
    n_chunks = per_w // CHUNK
    n_groups = n_chunks // K
    mesh = plsc.VectorSubcoreMesh(core_axis_name="c", subcore_axis_name="s")

    scratch = [
        pltpu.VMEM((n_chunks, CHUNK), jnp.int32),
        pltpu.VMEM((K * CHUNK, EMBED_DIM), jnp.float32),
        pltpu.VMEM((K * CHUNK, EMBED_DIM), jnp.float32),
    ] + [pltpu.SemaphoreType.DMA] * (2 * K + 2)

    @functools.partial(
        pl.kernel,
        out_type=jax.ShapeDtypeStruct((n_total, EMBED_DIM), jnp.float32),
        mesh=mesh,
        scratch_types=scratch,
        compiler_params=pltpu.CompilerParams(use_tc_tiling_on_sc=False),
    )
    def k(idx_hbm, table_hbm, out_hbm, idx_v, rows0, rows1, *sems):
        semg = (sems[:K], sems[K:2 * K])
        sems_store = sems[2 * K:]
        rows = (rows0, rows1)
        wid = lax.axis_index("s") * nc + lax.axis_index("c")
        row0 = wid * n_chunks
        pltpu.sync_copy(idx_hbm.at[pl.ds(row0, n_chunks)], idx_v)

        desc_g = [[None] * K for _ in range(2)]
        desc_s = [None, None]

        def fire_gathers(g, p):
            for kk in range(K):
                desc_g[p][kk] = pltpu.async_copy(
                    table_hbm.at[idx_v.at[g * K + kk]],
                    rows[p].at[pl.ds(kk * CHUNK, CHUNK)],
                    semg[p][kk],
                )

        fire_gathers(0, 0)
        for g in range(n_groups):
            p = g % 2
            pn = 1 - p
            if g + 1 < n_groups:
                if g >= 1:
                    desc_s[pn].wait()
                fire_gathers(g + 1, pn)
            for kk in range(K):
                desc_g[p][kk].wait()
            desc_s[p] = pltpu.async_copy(
                rows[p],
                out_hbm.at[pl.ds((row0 + g * K) * CHUNK, K * CHUNK)],
                sems_store[p],
            )
        desc_s[0].wait()
        desc_s[1].wait()

    return k


def kernel(velocity_bins, table):
    b, s = velocity_bins.shape
    n = b * s
    idx2d = velocity_bins.astype(jnp.int32).reshape(n // CHUNK, CHUNK)
    out = _sc_gather(n)(idx2d, table)
    return out.reshape(b, s, EMBED_DIM)


# speedup vs baseline: 3.4834x; 2.1288x over previous
import functools

import jax
import jax.numpy as jnp
from jax import lax
from jax.experimental import pallas as pl
from jax.experimental.pallas import tpu as pltpu
from jax.experimental.pallas import tpu_sc as plsc

NUM_BINS = 32
EMBED_DIM = 64
CHUNK = 128
K = 5


@functools.lru_cache(maxsize=None)
def _sc_gather(n_total: int):
    info = plsc.get_sparse_core_info()
    nc, ns = info.num_cores, info.num_subcores
    nw = nc * ns
    per_w = n_total // nw
    assert per_w * nw == n_total and per_w % (CHUNK * K) == 0
    n_chunks = per_w // CHUNK
    n_groups = n_chunks // K
    mesh = plsc.VectorSubcoreMesh(core_axis_name="c", subcore_axis_name="s")

    scratch = [
        pltpu.VMEM((n_chunks, CHUNK), jnp.int32),
        pltpu.VMEM((K * CHUNK, EMBED_DIM), jnp.float32),
        pltpu.VMEM((K * CHUNK, EMBED_DIM), jnp.float32),
    ] + [pltpu.SemaphoreType.DMA] * (2 * K + 2)

    @functools.partial(
        pl.kernel,
        out_type=jax.ShapeDtypeStruct((n_total, EMBED_DIM), jnp.float32),
        mesh=mesh,
        scratch_types=scratch,
        compiler_params=pltpu.CompilerParams(use_tc_tiling_on_sc=False),
    )
    def k(idx_hbm, table_hbm, out_hbm, idx_v, rows0, rows1, *sems):
        semg = (sems[:K], sems[K:2 * K])
        sems_store = sems[2 * K:]
        rows = (rows0, rows1)
        wid = lax.axis_index("s") * nc + lax.axis_index("c")
        row0 = wid * n_chunks
        pltpu.sync_copy(idx_hbm.at[pl.ds(row0, n_chunks)], idx_v)

        desc_g = [[None] * K for _ in range(2)]
        desc_s = [None, None]

        def fire_gathers(g, p):
            for kk in range(K):
                desc_g[p][kk] = pltpu.async_copy(
                    table_hbm.at[idx_v.at[g * K + kk]],
                    rows[p].at[pl.ds(kk * CHUNK, CHUNK)],
                    semg[p][kk],
                )

        fire_gathers(0, 0)
        for g in range(n_groups):
            p = g % 2
            pn = 1 - p
            if g + 1 < n_groups:
                if g >= 1:
                    desc_s[pn].wait()
                fire_gathers(g + 1, pn)
            for kk in range(K):
                desc_g[p][kk].wait()
            desc_s[p] = pltpu.async_copy(
                rows[p],
                out_hbm.at[pl.ds((row0 + g * K) * CHUNK, K * CHUNK)],
                sems_store[p],
            )
        desc_s[0].wait()
        desc_s[1].wait()

    return k


def kernel(velocity_bins, table):
    b, s = velocity_bins.shape
    n = b * s
    info = plsc.get_sparse_core_info()
    nw = info.num_cores * info.num_subcores
    idx2d = velocity_bins.astype(jnp.int32).reshape(n // CHUNK, CHUNK)
    chunks_per_w = (n // CHUNK) // nw
    offs = (jnp.arange(n // CHUNK, dtype=jnp.int32) // chunks_per_w) * NUM_BINS
    idx2d = idx2d + offs[:, None]
    table_rep = jnp.tile(table, (nw, 1))
    out = _sc_gather(n)(idx2d, table_rep)
    return out.reshape(b, s, EMBED_DIM)
